# baseline (speedup 1.0000x reference)
import functools
import jax
import jax.numpy as jnp
from jax import lax
from jax.experimental import pallas as pl
from jax.experimental.pallas import tpu as pltpu
from jax.experimental.pallas import tpu_sc as plsc

RB = 512
SC_CH = 128
SC_NW = 32


def _ln_gelu(h, g, be):
    mu = jnp.mean(h, axis=-1, keepdims=True)
    var = jnp.mean((h - mu) ** 2, axis=-1, keepdims=True)
    return jax.nn.gelu((h - mu) / jnp.sqrt(var + 1e-5) * g + be)


def _row_spec(C):
    return pl.BlockSpec((RB, C), lambda i: (i, 0))


def _full_spec(shape):
    return pl.BlockSpec(shape, lambda i: (0,) * len(shape))


def _stem_body(x_ref, W1_ref, b1_ref, g1_ref, be1_ref, W2_ref, b2_ref, g2_ref,
               be2_ref, o_ref):
    h = jnp.dot(x_ref[...], W1_ref[...], preferred_element_type=jnp.float32)
    h = _ln_gelu(h + b1_ref[...], g1_ref[...], be1_ref[...])
    h = jnp.dot(h, W2_ref[...], preferred_element_type=jnp.float32)
    o_ref[...] = _ln_gelu(h + b2_ref[...], g2_ref[...], be2_ref[...])


def _stem(x, W1, b1, g1, be1, W2, b2, g2, be2):
    N = x.shape[0]
    Co = W2.shape[1]
    args = [x, W1, b1.reshape(1, -1), g1.reshape(1, -1), be1.reshape(1, -1),
            W2, b2.reshape(1, -1), g2.reshape(1, -1), be2.reshape(1, -1)]
    return pl.pallas_call(
        _stem_body,
        grid=(N // RB,),
        in_specs=[_row_spec(x.shape[1])] + [_full_spec(a.shape) for a in args[1:]],
        out_specs=_row_spec(Co),
        out_shape=jax.ShapeDtypeStruct((N, Co), jnp.float32),
    )(*args)


def _cb_body(x_ref, n4_ref, Ws_ref, Wn_ref, b_ref, g_ref, be_ref, o_ref):
    C = x_ref.shape[1]
    n4 = n4_ref[...]
    nbrm = 0.25 * (n4[:, :C] + n4[:, C:2 * C] + n4[:, 2 * C:3 * C] + n4[:, 3 * C:])
    h = jnp.dot(x_ref[...], Ws_ref[...], preferred_element_type=jnp.float32)
    h = h + jnp.dot(nbrm, Wn_ref[...], preferred_element_type=jnp.float32)
    o_ref[...] = _ln_gelu(h + b_ref[...], g_ref[...], be_ref[...])


def _cb(x, nbr4, Ws, Wn, b, g, be):
    N = x.shape[0]
    Co = Ws.shape[1]
    args = [x, nbr4, Ws, Wn, b.reshape(1, -1), g.reshape(1, -1), be.reshape(1, -1)]
    return pl.pallas_call(
        _cb_body,
        grid=(N // RB,),
        in_specs=[_row_spec(x.shape[1]), _row_spec(nbr4.shape[1])]
        + [_full_spec(a.shape) for a in args[2:]],
        out_specs=_row_spec(Co),
        out_shape=jax.ShapeDtypeStruct((N, Co), jnp.float32),
    )(*args)


def _proj_body(x_ref, W_ref, o_ref):
    o_ref[...] = jnp.dot(x_ref[...], W_ref[...], preferred_element_type=jnp.float32)


def _proj(x, W):
    N = x.shape[0]
    Co = W.shape[1]
    return pl.pallas_call(
        _proj_body,
        grid=(N // RB,),
        in_specs=[_row_spec(x.shape[1]), _full_spec(W.shape)],
        out_specs=_row_spec(Co),
        out_shape=jax.ShapeDtypeStruct((N, Co), jnp.float32),
    )(x, W)


def _lin_body(x_ref, W_ref, a_ref, o_ref):
    o_ref[...] = a_ref[...] + jnp.dot(x_ref[...], W_ref[...],
                                      preferred_element_type=jnp.float32)


def _lin(x, W, a):
    N = x.shape[0]
    Co = W.shape[1]
    return pl.pallas_call(
        _lin_body,
        grid=(N // RB,),
        in_specs=[_row_spec(x.shape[1]), _full_spec(W.shape), _row_spec(Co)],
        out_specs=_row_spec(Co),
        out_shape=jax.ShapeDtypeStruct((N, Co), jnp.float32),
    )(x, W, a)


def _sum4(n4, Co):
    Cp = n4.shape[1] // 4
    return (n4[:, :Co] + n4[:, Cp:Cp + Co] + n4[:, 2 * Cp:2 * Cp + Co]
            + n4[:, 3 * Cp:3 * Cp + Co])


def _cbp_body(x_ref, n4_ref, Ws_ref, b_ref, g_ref, be_ref, o_ref):
    h = jnp.dot(x_ref[...], Ws_ref[...], preferred_element_type=jnp.float32)
    h = h + 0.25 * _sum4(n4_ref[...], Ws_ref.shape[1]) + b_ref[...]
    o_ref[...] = _ln_gelu(h, g_ref[...], be_ref[...])


def _cbp(x, nbr4, Ws, b, g, be):
    N = x.shape[0]
    Co = Ws.shape[1]
    args = [x, nbr4, Ws, b.reshape(1, -1), g.reshape(1, -1), be.reshape(1, -1)]
    return pl.pallas_call(
        _cbp_body,
        grid=(N // RB,),
        in_specs=[_row_spec(x.shape[1]), _row_spec(nbr4.shape[1])]
        + [_full_spec(a.shape) for a in args[2:]],
        out_specs=_row_spec(Co),
        out_shape=jax.ShapeDtypeStruct((N, Co), jnp.float32),
    )(*args)


def _fin_body(x_ref, W_ref, a1_ref, n4_ref, b_ref, g_ref, be_ref, o_ref):
    Co = W_ref.shape[1]
    nbrm = 0.25 * _sum4(n4_ref[...], Co)
    h = jnp.dot(x_ref[...], W_ref[...], preferred_element_type=jnp.float32)
    h = h + a1_ref[...] + nbrm + b_ref[...]
    o_ref[...] = _ln_gelu(h, g_ref[...], be_ref[...])


def _fin(x, W, a1, nbr4, b, g, be):
    N = x.shape[0]
    Co = W.shape[1]
    args = [x, W, a1, nbr4, b.reshape(1, -1), g.reshape(1, -1), be.reshape(1, -1)]
    return pl.pallas_call(
        _fin_body,
        grid=(N // RB,),
        in_specs=[_row_spec(x.shape[1]), _full_spec(W.shape), _row_spec(Co),
                  _row_spec(nbr4.shape[1])] + [_full_spec(a.shape) for a in args[4:]],
        out_specs=_row_spec(Co),
        out_shape=jax.ShapeDtypeStruct((N, Co), jnp.float32),
    )(*args)


def _head_body(x_ref, W1_ref, b1_ref, W2_ref, b2_ref, o_ref):
    h = jnp.dot(x_ref[...], W1_ref[...], preferred_element_type=jnp.float32)
    h = jax.nn.gelu(h + b1_ref[...])
    o_ref[...] = jnp.dot(h, W2_ref[...],
                         preferred_element_type=jnp.float32) + b2_ref[...]


def _head(x, W1, b1, W2, b2):
    N = x.shape[0]
    args = [x, W1, b1.reshape(1, -1), W2, b2.reshape(1, -1)]
    return pl.pallas_call(
        _head_body,
        grid=(N // RB,),
        in_specs=[_row_spec(x.shape[1])] + [_full_spec(a.shape) for a in args[1:]],
        out_specs=_row_spec(1),
        out_shape=jax.ShapeDtypeStruct((N, 1), jnp.float32),
    )(*args)


def _sc_gather(table, idx):
    C = table.shape[1]
    K = max(1, 768 // C)
    gran = SC_NW * SC_CH * K
    Mp = _rup(idx.shape[0], gran)
    idx = jnp.pad(idx, (0, Mp - idx.shape[0]))
    per_w = Mp // SC_NW
    iters = per_w // (SC_CH * K)

    @functools.partial(
        pl.kernel,
        mesh=plsc.VectorSubcoreMesh(core_axis_name="c", subcore_axis_name="s"),
        out_type=jax.ShapeDtypeStruct((Mp, C), jnp.float32),
        scratch_types=[
            pltpu.VMEM((K, SC_CH), jnp.int32),
            pltpu.VMEM((K, SC_CH, C), jnp.float32),
            pltpu.SemaphoreType.DMA,
            pltpu.SemaphoreType.DMA,
            pltpu.SemaphoreType.DMA,
        ],
    )
    def k(table_hbm, idx_hbm, out_hbm, idx_v, rows_v, sem_i, sem_g, sem_s):
        wid = lax.axis_index("s") * 2 + lax.axis_index("c")
        base0 = wid * per_w

        def body(j, carry):
            base = base0 + j * SC_CH * K
            loads = [
                pltpu.async_copy(idx_hbm.at[pl.ds(base + b * SC_CH, SC_CH)],
                                 idx_v.at[b], sem_i)
                for b in range(K)
            ]
            for cp in loads:
                cp.wait()
            gets = [
                pltpu.async_copy(table_hbm.at[idx_v.at[b]], rows_v.at[b], sem_g)
                for b in range(K)
            ]
            for cp in gets:
                cp.wait()
            puts = [
                pltpu.async_copy(rows_v.at[b],
                                 out_hbm.at[pl.ds(base + b * SC_CH, SC_CH)],
                                 sem_s)
                for b in range(K)
            ]
            for cp in puts:
                cp.wait()
            return carry

        lax.fori_loop(0, iters, body, 0)

    return k(table, idx)


def _gather4(x, adj_flat, n_real):
    g = _sc_gather(x, adj_flat)
    C = x.shape[1]
    return g[:4 * n_real].reshape(n_real, 4 * C)


def _gather_rows(x, idx, n_real):
    return _sc_gather(x, idx)[:n_real]


def _seg_mean(h, pmap, n_out):
    sums = jax.ops.segment_sum(h, pmap, num_segments=n_out)
    cnt = jax.ops.segment_sum(jnp.ones((h.shape[0],), h.dtype), pmap,
                              num_segments=n_out)
    return sums / jnp.maximum(cnt, 1.0)[:, None]


def _pad_rows(a, n, val=0):
    return jnp.pad(a, ((0, n - a.shape[0]),) + ((0, 0),) * (a.ndim - 1),
                   constant_values=val)


def _rup(n, m=RB):
    return ((n + m - 1) // m) * m


_SC_M = SC_NW * SC_CH


def _pad_idx(a, val=0):
    return _pad_rows(a.reshape(-1, 1), _rup(a.shape[0], _SC_M), val).reshape(-1)


def kernel(edge_features, slot_adj_l0, slot_adj_l1, slot_adj_l2, pool_map_l0,
           pool_map_l1, stem_W1, stem_b1, stem_g1, stem_be1, stem_W2, stem_b2,
           stem_g2, stem_be2, enc0_Ws, enc0_Wn, enc0_b, enc0_g, enc0_be,
           enc1_Ws, enc1_Wn, enc1_b, enc1_g, enc1_be, bot_Ws, bot_Wn, bot_b,
           bot_g, bot_be, dec0_Ws, dec0_Wn, dec0_b, dec0_g, dec0_be, dec1_Ws,
           dec1_Wn, dec1_b, dec1_g, dec1_be, head_W1, head_b1, head_W2,
           head_b2):
    E0, E1, E2 = edge_features.shape[0], slot_adj_l1.shape[0], slot_adj_l2.shape[0]
    C1, C2 = enc1_Ws.shape[1], bot_Ws.shape[1]
    E0p, E1p, E2p = _rup(E0), _rup(E1), _rup(E2)

    x = _pad_rows(edge_features, E0p)
    adj0f = _pad_idx(_pad_rows(slot_adj_l0, E0p).reshape(-1))
    adj1f = _pad_idx(_pad_rows(slot_adj_l1, E1p).reshape(-1))
    adj2f = _pad_idx(_pad_rows(slot_adj_l2, E2p).reshape(-1))
    pmap0 = _pad_rows(pool_map_l0.reshape(-1, 1), E0p, E1).reshape(-1)
    pmap1 = _pad_rows(pool_map_l1.reshape(-1, 1), E1p, E2).reshape(-1)
    pmap0g = _pad_idx(pmap0)
    pmap1g = _pad_idx(pmap1)

    h = _stem(x, stem_W1, stem_b1, stem_g1, stem_be1, stem_W2, stem_b2,
              stem_g2, stem_be2)
    h = _cb(h, _gather4(h, adj0f, E0p), enc0_Ws, enc0_Wn, enc0_b, enc0_g, enc0_be)
    skip0 = h
    h = _seg_mean(h, pmap0, E1p)
    h = _cb(h, _gather4(h, adj1f, E1p), enc1_Ws, enc1_Wn, enc1_b, enc1_g, enc1_be)
    skip1 = h
    h = _seg_mean(h, pmap1, E2p)
    z = _proj(h, bot_Wn)
    h = _cbp(h, _gather4(z, adj2f, E2p), bot_Ws, bot_b, bot_g, bot_be)

    W_up = jnp.concatenate([dec0_Ws[:C2], dec0_Wn[:C2]], axis=1)
    gAB = _gather_rows(_proj(h, W_up), pmap1g, E1p)
    y = _lin(skip1, dec0_Wn[C2:], gAB[:, C1:])
    y = jnp.pad(y, ((0, 0), (0, _rup(C1, 128) - C1)))
    h = _fin(skip1, dec0_Ws[C2:], gAB[:, :C1], _gather4(y, adj1f, E1p),
             dec0_b, dec0_g, dec0_be)

    C0 = dec1_Ws.shape[1]
    W_up0 = jnp.concatenate([dec1_Ws[:C1], dec1_Wn[:C1]], axis=1)
    gAB0 = _gather_rows(_proj(h, W_up0), pmap0g, E0p)
    y0 = _lin(skip0, dec1_Wn[C1:], gAB0[:, C0:])
    h = _fin(skip0, dec1_Ws[C1:], gAB0[:, :C0], _gather4(y0, adj0f, E0p),
             dec1_b, dec1_g, dec1_be)

    out = _head(h, head_W1, head_b1, head_W2, head_b2)
    return out[:E0, 0]


# speedup vs baseline: 1.2376x; 1.2376x over previous
import functools
import jax
import jax.numpy as jnp
from jax import lax
from jax.experimental import pallas as pl
from jax.experimental.pallas import tpu as pltpu
from jax.experimental.pallas import tpu_sc as plsc

RB = 512
SC_CH = 128
SC_NW = 32


def _ln_gelu(h, g, be):
    mu = jnp.mean(h, axis=-1, keepdims=True)
    var = jnp.mean((h - mu) ** 2, axis=-1, keepdims=True)
    return jax.nn.gelu((h - mu) / jnp.sqrt(var + 1e-5) * g + be)


def _row_spec(C):
    return pl.BlockSpec((RB, C), lambda i: (i, 0))


def _full_spec(shape):
    return pl.BlockSpec(shape, lambda i: (0,) * len(shape))


def _stem_body(x_ref, W1_ref, b1_ref, g1_ref, be1_ref, W2_ref, b2_ref, g2_ref,
               be2_ref, o_ref):
    h = jnp.dot(x_ref[...], W1_ref[...], preferred_element_type=jnp.float32)
    h = _ln_gelu(h + b1_ref[...], g1_ref[...], be1_ref[...])
    h = jnp.dot(h, W2_ref[...], preferred_element_type=jnp.float32)
    o_ref[...] = _ln_gelu(h + b2_ref[...], g2_ref[...], be2_ref[...])


def _stem(x, W1, b1, g1, be1, W2, b2, g2, be2):
    N = x.shape[0]
    Co = W2.shape[1]
    args = [x, W1, b1.reshape(1, -1), g1.reshape(1, -1), be1.reshape(1, -1),
            W2, b2.reshape(1, -1), g2.reshape(1, -1), be2.reshape(1, -1)]
    return pl.pallas_call(
        _stem_body,
        grid=(N // RB,),
        in_specs=[_row_spec(x.shape[1])] + [_full_spec(a.shape) for a in args[1:]],
        out_specs=_row_spec(Co),
        out_shape=jax.ShapeDtypeStruct((N, Co), jnp.float32),
    )(*args)


def _cb_body(x_ref, n4_ref, Ws_ref, Wn_ref, b_ref, g_ref, be_ref, o_ref):
    C = x_ref.shape[1]
    n4 = n4_ref[...]
    nbrm = 0.25 * (n4[:, :C] + n4[:, C:2 * C] + n4[:, 2 * C:3 * C] + n4[:, 3 * C:])
    h = jnp.dot(x_ref[...], Ws_ref[...], preferred_element_type=jnp.float32)
    h = h + jnp.dot(nbrm, Wn_ref[...], preferred_element_type=jnp.float32)
    o_ref[...] = _ln_gelu(h + b_ref[...], g_ref[...], be_ref[...])


def _cb(x, nbr4, Ws, Wn, b, g, be):
    N = x.shape[0]
    Co = Ws.shape[1]
    args = [x, nbr4, Ws, Wn, b.reshape(1, -1), g.reshape(1, -1), be.reshape(1, -1)]
    return pl.pallas_call(
        _cb_body,
        grid=(N // RB,),
        in_specs=[_row_spec(x.shape[1]), _row_spec(nbr4.shape[1])]
        + [_full_spec(a.shape) for a in args[2:]],
        out_specs=_row_spec(Co),
        out_shape=jax.ShapeDtypeStruct((N, Co), jnp.float32),
    )(*args)


def _proj_body(x_ref, W_ref, o_ref):
    o_ref[...] = jnp.dot(x_ref[...], W_ref[...], preferred_element_type=jnp.float32)


def _proj(x, W):
    N = x.shape[0]
    Co = W.shape[1]
    return pl.pallas_call(
        _proj_body,
        grid=(N // RB,),
        in_specs=[_row_spec(x.shape[1]), _full_spec(W.shape)],
        out_specs=_row_spec(Co),
        out_shape=jax.ShapeDtypeStruct((N, Co), jnp.float32),
    )(x, W)


def _lin_body(x_ref, W_ref, a_ref, o_ref):
    o_ref[...] = a_ref[...] + jnp.dot(x_ref[...], W_ref[...],
                                      preferred_element_type=jnp.float32)


def _lin(x, W, a):
    N = x.shape[0]
    Co = W.shape[1]
    return pl.pallas_call(
        _lin_body,
        grid=(N // RB,),
        in_specs=[_row_spec(x.shape[1]), _full_spec(W.shape), _row_spec(Co)],
        out_specs=_row_spec(Co),
        out_shape=jax.ShapeDtypeStruct((N, Co), jnp.float32),
    )(x, W, a)


def _sum4(n4, Co):
    Cp = n4.shape[1] // 4
    return (n4[:, :Co] + n4[:, Cp:Cp + Co] + n4[:, 2 * Cp:2 * Cp + Co]
            + n4[:, 3 * Cp:3 * Cp + Co])


def _cbp_body(x_ref, n4_ref, Ws_ref, b_ref, g_ref, be_ref, o_ref):
    h = jnp.dot(x_ref[...], Ws_ref[...], preferred_element_type=jnp.float32)
    h = h + 0.25 * _sum4(n4_ref[...], Ws_ref.shape[1]) + b_ref[...]
    o_ref[...] = _ln_gelu(h, g_ref[...], be_ref[...])


def _cbp(x, nbr4, Ws, b, g, be):
    N = x.shape[0]
    Co = Ws.shape[1]
    args = [x, nbr4, Ws, b.reshape(1, -1), g.reshape(1, -1), be.reshape(1, -1)]
    return pl.pallas_call(
        _cbp_body,
        grid=(N // RB,),
        in_specs=[_row_spec(x.shape[1]), _row_spec(nbr4.shape[1])]
        + [_full_spec(a.shape) for a in args[2:]],
        out_specs=_row_spec(Co),
        out_shape=jax.ShapeDtypeStruct((N, Co), jnp.float32),
    )(*args)


def _fin_body(x_ref, W_ref, a1_ref, n4_ref, b_ref, g_ref, be_ref, o_ref):
    Co = W_ref.shape[1]
    nbrm = 0.25 * _sum4(n4_ref[...], Co)
    h = jnp.dot(x_ref[...], W_ref[...], preferred_element_type=jnp.float32)
    h = h + a1_ref[...] + nbrm + b_ref[...]
    o_ref[...] = _ln_gelu(h, g_ref[...], be_ref[...])


def _fin(x, W, a1, nbr4, b, g, be):
    N = x.shape[0]
    Co = W.shape[1]
    args = [x, W, a1, nbr4, b.reshape(1, -1), g.reshape(1, -1), be.reshape(1, -1)]
    return pl.pallas_call(
        _fin_body,
        grid=(N // RB,),
        in_specs=[_row_spec(x.shape[1]), _full_spec(W.shape), _row_spec(Co),
                  _row_spec(nbr4.shape[1])] + [_full_spec(a.shape) for a in args[4:]],
        out_specs=_row_spec(Co),
        out_shape=jax.ShapeDtypeStruct((N, Co), jnp.float32),
    )(*args)


def _head_body(x_ref, W1_ref, b1_ref, W2_ref, b2_ref, o_ref):
    h = jnp.dot(x_ref[...], W1_ref[...], preferred_element_type=jnp.float32)
    h = jax.nn.gelu(h + b1_ref[...])
    o_ref[...] = jnp.dot(h, W2_ref[...],
                         preferred_element_type=jnp.float32) + b2_ref[...]


def _head(x, W1, b1, W2, b2):
    N = x.shape[0]
    args = [x, W1, b1.reshape(1, -1), W2, b2.reshape(1, -1)]
    return pl.pallas_call(
        _head_body,
        grid=(N // RB,),
        in_specs=[_row_spec(x.shape[1])] + [_full_spec(a.shape) for a in args[1:]],
        out_specs=_row_spec(1),
        out_shape=jax.ShapeDtypeStruct((N, 1), jnp.float32),
    )(*args)


def _sc_gather(table, idx):
    C = table.shape[1]
    gran = SC_NW * SC_CH
    Mp = _rup(idx.shape[0], gran)
    idx = jnp.pad(idx, (0, Mp - idx.shape[0]))
    per_w = Mp // SC_NW
    iters = per_w // SC_CH

    @functools.partial(
        pl.kernel,
        mesh=plsc.VectorSubcoreMesh(core_axis_name="c", subcore_axis_name="s"),
        out_type=jax.ShapeDtypeStruct((Mp, C), jnp.float32),
        scratch_types=[
            pltpu.VMEM((SC_CH,), jnp.int32),
            pltpu.VMEM((SC_CH, C), jnp.float32),
            pltpu.SemaphoreType.DMA,
        ],
    )
    def k(table_hbm, idx_hbm, out_hbm, idx_v, rows_v, sem):
        wid = lax.axis_index("s") * 2 + lax.axis_index("c")
        base0 = wid * per_w

        def body(j, carry):
            base = base0 + j * SC_CH
            pltpu.sync_copy(idx_hbm.at[pl.ds(base, SC_CH)], idx_v)
            pltpu.async_copy(table_hbm.at[idx_v], rows_v, sem).wait()
            pltpu.sync_copy(rows_v, out_hbm.at[pl.ds(base, SC_CH)])
            return carry

        lax.fori_loop(0, iters, body, 0)

    return k(table, idx)


def _gather4(x, adj_flat, n_real):
    g = _sc_gather(x, adj_flat)
    C = x.shape[1]
    return g[:4 * n_real].reshape(n_real, 4 * C)


def _gather_rows(x, idx, n_real):
    return _sc_gather(x, idx)[:n_real]


def _seg_mean(h, pmap, n_out):
    sums = jax.ops.segment_sum(h, pmap, num_segments=n_out)
    cnt = jax.ops.segment_sum(jnp.ones((h.shape[0],), h.dtype), pmap,
                              num_segments=n_out)
    return sums / jnp.maximum(cnt, 1.0)[:, None]


def _pad_rows(a, n, val=0):
    return jnp.pad(a, ((0, n - a.shape[0]),) + ((0, 0),) * (a.ndim - 1),
                   constant_values=val)


def _rup(n, m=RB):
    return ((n + m - 1) // m) * m


_SC_M = SC_NW * SC_CH


def _pad_idx(a, val=0):
    return _pad_rows(a.reshape(-1, 1), _rup(a.shape[0], _SC_M), val).reshape(-1)


def kernel(edge_features, slot_adj_l0, slot_adj_l1, slot_adj_l2, pool_map_l0,
           pool_map_l1, stem_W1, stem_b1, stem_g1, stem_be1, stem_W2, stem_b2,
           stem_g2, stem_be2, enc0_Ws, enc0_Wn, enc0_b, enc0_g, enc0_be,
           enc1_Ws, enc1_Wn, enc1_b, enc1_g, enc1_be, bot_Ws, bot_Wn, bot_b,
           bot_g, bot_be, dec0_Ws, dec0_Wn, dec0_b, dec0_g, dec0_be, dec1_Ws,
           dec1_Wn, dec1_b, dec1_g, dec1_be, head_W1, head_b1, head_W2,
           head_b2):
    E0, E1, E2 = edge_features.shape[0], slot_adj_l1.shape[0], slot_adj_l2.shape[0]
    C1, C2 = enc1_Ws.shape[1], bot_Ws.shape[1]
    E0p, E1p, E2p = _rup(E0), _rup(E1), _rup(E2)

    x = _pad_rows(edge_features, E0p)
    adj0f = _pad_idx(_pad_rows(slot_adj_l0, E0p).reshape(-1))
    adj1f = _pad_idx(_pad_rows(slot_adj_l1, E1p).reshape(-1))
    adj2f = _pad_idx(_pad_rows(slot_adj_l2, E2p).reshape(-1))
    pmap0 = _pad_rows(pool_map_l0.reshape(-1, 1), E0p, E1).reshape(-1)
    pmap1 = _pad_rows(pool_map_l1.reshape(-1, 1), E1p, E2).reshape(-1)
    pmap0g = _pad_idx(pmap0)
    pmap1g = _pad_idx(pmap1)

    h = _stem(x, stem_W1, stem_b1, stem_g1, stem_be1, stem_W2, stem_b2,
              stem_g2, stem_be2)
    h = _cb(h, _gather4(h, adj0f, E0p), enc0_Ws, enc0_Wn, enc0_b, enc0_g, enc0_be)
    skip0 = h
    h = _seg_mean(h, pmap0, E1p)
    h = _cb(h, _gather4(h, adj1f, E1p), enc1_Ws, enc1_Wn, enc1_b, enc1_g, enc1_be)
    skip1 = h
    h = _seg_mean(h, pmap1, E2p)
    z = _proj(h, bot_Wn)
    h = _cbp(h, _gather4(z, adj2f, E2p), bot_Ws, bot_b, bot_g, bot_be)

    W_up = jnp.concatenate([dec0_Ws[:C2], dec0_Wn[:C2]], axis=1)
    gAB = _gather_rows(_proj(h, W_up), pmap1g, E1p)
    y = _lin(skip1, dec0_Wn[C2:], gAB[:, C1:])
    y = jnp.pad(y, ((0, 0), (0, _rup(C1, 128) - C1)))
    h = _fin(skip1, dec0_Ws[C2:], gAB[:, :C1], _gather4(y, adj1f, E1p),
             dec0_b, dec0_g, dec0_be)

    C0 = dec1_Ws.shape[1]
    W_up0 = jnp.concatenate([dec1_Ws[:C1], dec1_Wn[:C1]], axis=1)
    gAB0 = _gather_rows(_proj(h, W_up0), pmap0g, E0p)
    y0 = _lin(skip0, dec1_Wn[C1:], gAB0[:, C0:])
    h = _fin(skip0, dec1_Ws[C1:], gAB0[:, :C0], _gather4(y0, adj0f, E0p),
             dec1_b, dec1_g, dec1_be)

    out = _head(h, head_W1, head_b1, head_W2, head_b2)
    return out[:E0, 0]


# speedup vs baseline: 1.2396x; 1.0016x over previous
import functools
import jax
import jax.numpy as jnp
from jax import lax
from jax.experimental import pallas as pl
from jax.experimental.pallas import tpu as pltpu
from jax.experimental.pallas import tpu_sc as plsc

RB = 512
SC_CH = 128
SC_NW = 32


def _ln_gelu(h, g, be):
    mu = jnp.mean(h, axis=-1, keepdims=True)
    var = jnp.mean((h - mu) ** 2, axis=-1, keepdims=True)
    return jax.nn.gelu((h - mu) / jnp.sqrt(var + 1e-5) * g + be)


def _row_spec(C):
    return pl.BlockSpec((RB, C), lambda i: (i, 0))


def _full_spec(shape):
    return pl.BlockSpec(shape, lambda i: (0,) * len(shape))


def _stem_body(x_ref, W1_ref, b1_ref, g1_ref, be1_ref, W2_ref, b2_ref, g2_ref,
               be2_ref, o_ref):
    h = jnp.dot(x_ref[...], W1_ref[...], preferred_element_type=jnp.float32)
    h = _ln_gelu(h + b1_ref[...], g1_ref[...], be1_ref[...])
    h = jnp.dot(h, W2_ref[...], preferred_element_type=jnp.float32)
    o_ref[...] = _ln_gelu(h + b2_ref[...], g2_ref[...], be2_ref[...])


def _stem(x, W1, b1, g1, be1, W2, b2, g2, be2):
    N = x.shape[0]
    Co = W2.shape[1]
    args = [x, W1, b1.reshape(1, -1), g1.reshape(1, -1), be1.reshape(1, -1),
            W2, b2.reshape(1, -1), g2.reshape(1, -1), be2.reshape(1, -1)]
    return pl.pallas_call(
        _stem_body,
        grid=(N // RB,),
        in_specs=[_row_spec(x.shape[1])] + [_full_spec(a.shape) for a in args[1:]],
        out_specs=_row_spec(Co),
        out_shape=jax.ShapeDtypeStruct((N, Co), jnp.float32),
    )(*args)


def _cb_body(x_ref, n4_ref, Ws_ref, Wn_ref, b_ref, g_ref, be_ref, o_ref):
    C = x_ref.shape[1]
    n4 = n4_ref[...]
    nbrm = 0.25 * (n4[:, :C] + n4[:, C:2 * C] + n4[:, 2 * C:3 * C] + n4[:, 3 * C:])
    h = jnp.dot(x_ref[...], Ws_ref[...], preferred_element_type=jnp.float32)
    h = h + jnp.dot(nbrm, Wn_ref[...], preferred_element_type=jnp.float32)
    o_ref[...] = _ln_gelu(h + b_ref[...], g_ref[...], be_ref[...])


def _cb(x, nbr4, Ws, Wn, b, g, be):
    N = x.shape[0]
    Co = Ws.shape[1]
    args = [x, nbr4, Ws, Wn, b.reshape(1, -1), g.reshape(1, -1), be.reshape(1, -1)]
    return pl.pallas_call(
        _cb_body,
        grid=(N // RB,),
        in_specs=[_row_spec(x.shape[1]), _row_spec(nbr4.shape[1])]
        + [_full_spec(a.shape) for a in args[2:]],
        out_specs=_row_spec(Co),
        out_shape=jax.ShapeDtypeStruct((N, Co), jnp.float32),
    )(*args)


def _proj_body(x_ref, W_ref, o_ref):
    o_ref[...] = jnp.dot(x_ref[...], W_ref[...], preferred_element_type=jnp.float32)


def _proj(x, W):
    N = x.shape[0]
    Co = W.shape[1]
    return pl.pallas_call(
        _proj_body,
        grid=(N // RB,),
        in_specs=[_row_spec(x.shape[1]), _full_spec(W.shape)],
        out_specs=_row_spec(Co),
        out_shape=jax.ShapeDtypeStruct((N, Co), jnp.float32),
    )(x, W)


def _lin_body(x_ref, W_ref, a_ref, o_ref):
    o_ref[...] = a_ref[...] + jnp.dot(x_ref[...], W_ref[...],
                                      preferred_element_type=jnp.float32)


def _lin(x, W, a):
    N = x.shape[0]
    Co = W.shape[1]
    return pl.pallas_call(
        _lin_body,
        grid=(N // RB,),
        in_specs=[_row_spec(x.shape[1]), _full_spec(W.shape), _row_spec(Co)],
        out_specs=_row_spec(Co),
        out_shape=jax.ShapeDtypeStruct((N, Co), jnp.float32),
    )(x, W, a)


def _sum4(n4, Co):
    Cp = n4.shape[1] // 4
    return (n4[:, :Co] + n4[:, Cp:Cp + Co] + n4[:, 2 * Cp:2 * Cp + Co]
            + n4[:, 3 * Cp:3 * Cp + Co])


def _cbp_body(x_ref, n4_ref, Ws_ref, b_ref, g_ref, be_ref, o_ref):
    h = jnp.dot(x_ref[...], Ws_ref[...], preferred_element_type=jnp.float32)
    h = h + 0.25 * _sum4(n4_ref[...], Ws_ref.shape[1]) + b_ref[...]
    o_ref[...] = _ln_gelu(h, g_ref[...], be_ref[...])


def _cbp(x, nbr4, Ws, b, g, be):
    N = x.shape[0]
    Co = Ws.shape[1]
    args = [x, nbr4, Ws, b.reshape(1, -1), g.reshape(1, -1), be.reshape(1, -1)]
    return pl.pallas_call(
        _cbp_body,
        grid=(N // RB,),
        in_specs=[_row_spec(x.shape[1]), _row_spec(nbr4.shape[1])]
        + [_full_spec(a.shape) for a in args[2:]],
        out_specs=_row_spec(Co),
        out_shape=jax.ShapeDtypeStruct((N, Co), jnp.float32),
    )(*args)


def _fin_body(x_ref, W_ref, a1_ref, n4_ref, b_ref, g_ref, be_ref, o_ref):
    Co = W_ref.shape[1]
    nbrm = 0.25 * _sum4(n4_ref[...], Co)
    h = jnp.dot(x_ref[...], W_ref[...], preferred_element_type=jnp.float32)
    h = h + a1_ref[...] + nbrm + b_ref[...]
    o_ref[...] = _ln_gelu(h, g_ref[...], be_ref[...])


def _fin(x, W, a1, nbr4, b, g, be):
    N = x.shape[0]
    Co = W.shape[1]
    args = [x, W, a1, nbr4, b.reshape(1, -1), g.reshape(1, -1), be.reshape(1, -1)]
    return pl.pallas_call(
        _fin_body,
        grid=(N // RB,),
        in_specs=[_row_spec(x.shape[1]), _full_spec(W.shape), _row_spec(Co),
                  _row_spec(nbr4.shape[1])] + [_full_spec(a.shape) for a in args[4:]],
        out_specs=_row_spec(Co),
        out_shape=jax.ShapeDtypeStruct((N, Co), jnp.float32),
    )(*args)


def _head_body(x_ref, W1_ref, b1_ref, W2_ref, b2_ref, o_ref):
    h = jnp.dot(x_ref[...], W1_ref[...], preferred_element_type=jnp.float32)
    h = jax.nn.gelu(h + b1_ref[...])
    o_ref[...] = jnp.dot(h, W2_ref[...],
                         preferred_element_type=jnp.float32) + b2_ref[...]


def _head(x, W1, b1, W2, b2):
    N = x.shape[0]
    args = [x, W1, b1.reshape(1, -1), W2, b2.reshape(1, -1)]
    return pl.pallas_call(
        _head_body,
        grid=(N // RB,),
        in_specs=[_row_spec(x.shape[1])] + [_full_spec(a.shape) for a in args[1:]],
        out_specs=_row_spec(1),
        out_shape=jax.ShapeDtypeStruct((N, 1), jnp.float32),
    )(*args)


def _sc_gather(table, idx):
    C = table.shape[1]
    gran = 2 * SC_NW * SC_CH
    Mp = _rup(idx.shape[0], gran)
    idx = jnp.pad(idx, (0, Mp + SC_NW * SC_CH - idx.shape[0]))
    per_w = Mp // SC_NW
    pairs = per_w // (2 * SC_CH)

    @functools.partial(
        pl.kernel,
        mesh=plsc.VectorSubcoreMesh(core_axis_name="c", subcore_axis_name="s"),
        out_type=jax.ShapeDtypeStruct((Mp, C), jnp.float32),
        scratch_types=[
            pltpu.VMEM((SC_CH,), jnp.int32),
            pltpu.VMEM((SC_CH,), jnp.int32),
            pltpu.VMEM((SC_CH, C), jnp.float32),
            pltpu.VMEM((SC_CH, C), jnp.float32),
            pltpu.SemaphoreType.DMA,
            pltpu.SemaphoreType.DMA,
        ],
    )
    def k(table_hbm, idx_hbm, out_hbm, ia, ib, ra, rb, sema, semb):
        wid = lax.axis_index("s") * 2 + lax.axis_index("c")
        base0 = wid * per_w

        pltpu.sync_copy(idx_hbm.at[pl.ds(base0, SC_CH)], ia)
        pltpu.async_copy(table_hbm.at[ia], ra, sema)

        def body(jj, carry):
            b0 = base0 + 2 * jj * SC_CH
            pltpu.sync_copy(idx_hbm.at[pl.ds(b0 + SC_CH, SC_CH)], ib)
            pltpu.async_copy(table_hbm.at[ib], rb, semb)
            pltpu.make_async_copy(table_hbm.at[ia], ra, sema).wait()
            pltpu.sync_copy(ra, out_hbm.at[pl.ds(b0, SC_CH)])
            pltpu.sync_copy(idx_hbm.at[pl.ds(b0 + 2 * SC_CH, SC_CH)], ia)
            pltpu.async_copy(table_hbm.at[ia], ra, sema)
            pltpu.make_async_copy(table_hbm.at[ib], rb, semb).wait()
            pltpu.sync_copy(rb, out_hbm.at[pl.ds(b0 + SC_CH, SC_CH)])
            return carry

        lax.fori_loop(0, pairs, body, 0)
        pltpu.make_async_copy(table_hbm.at[ia], ra, sema).wait()

    return k(table, idx)


def _gather4(x, adj_flat, n_real):
    g = _sc_gather(x, adj_flat)
    C = x.shape[1]
    return g[:4 * n_real].reshape(n_real, 4 * C)


def _gather_rows(x, idx, n_real):
    return _sc_gather(x, idx)[:n_real]


def _seg_mean(h, pmap, n_out):
    sums = jax.ops.segment_sum(h, pmap, num_segments=n_out)
    cnt = jax.ops.segment_sum(jnp.ones((h.shape[0],), h.dtype), pmap,
                              num_segments=n_out)
    return sums / jnp.maximum(cnt, 1.0)[:, None]


def _pad_rows(a, n, val=0):
    return jnp.pad(a, ((0, n - a.shape[0]),) + ((0, 0),) * (a.ndim - 1),
                   constant_values=val)


def _rup(n, m=RB):
    return ((n + m - 1) // m) * m


_SC_M = SC_NW * SC_CH


def _pad_idx(a, val=0):
    return _pad_rows(a.reshape(-1, 1), _rup(a.shape[0], _SC_M), val).reshape(-1)


def kernel(edge_features, slot_adj_l0, slot_adj_l1, slot_adj_l2, pool_map_l0,
           pool_map_l1, stem_W1, stem_b1, stem_g1, stem_be1, stem_W2, stem_b2,
           stem_g2, stem_be2, enc0_Ws, enc0_Wn, enc0_b, enc0_g, enc0_be,
           enc1_Ws, enc1_Wn, enc1_b, enc1_g, enc1_be, bot_Ws, bot_Wn, bot_b,
           bot_g, bot_be, dec0_Ws, dec0_Wn, dec0_b, dec0_g, dec0_be, dec1_Ws,
           dec1_Wn, dec1_b, dec1_g, dec1_be, head_W1, head_b1, head_W2,
           head_b2):
    E0, E1, E2 = edge_features.shape[0], slot_adj_l1.shape[0], slot_adj_l2.shape[0]
    C1, C2 = enc1_Ws.shape[1], bot_Ws.shape[1]
    E0p, E1p, E2p = _rup(E0), _rup(E1), _rup(E2)

    x = _pad_rows(edge_features, E0p)
    adj0f = _pad_idx(_pad_rows(slot_adj_l0, E0p).reshape(-1))
    adj1f = _pad_idx(_pad_rows(slot_adj_l1, E1p).reshape(-1))
    adj2f = _pad_idx(_pad_rows(slot_adj_l2, E2p).reshape(-1))
    pmap0 = _pad_rows(pool_map_l0.reshape(-1, 1), E0p, E1).reshape(-1)
    pmap1 = _pad_rows(pool_map_l1.reshape(-1, 1), E1p, E2).reshape(-1)
    pmap0g = _pad_idx(pmap0)
    pmap1g = _pad_idx(pmap1)

    h = _stem(x, stem_W1, stem_b1, stem_g1, stem_be1, stem_W2, stem_b2,
              stem_g2, stem_be2)
    h = _cb(h, _gather4(h, adj0f, E0p), enc0_Ws, enc0_Wn, enc0_b, enc0_g, enc0_be)
    skip0 = h
    h = _seg_mean(h, pmap0, E1p)
    h = _cb(h, _gather4(h, adj1f, E1p), enc1_Ws, enc1_Wn, enc1_b, enc1_g, enc1_be)
    skip1 = h
    h = _seg_mean(h, pmap1, E2p)
    z = _proj(h, bot_Wn)
    h = _cbp(h, _gather4(z, adj2f, E2p), bot_Ws, bot_b, bot_g, bot_be)

    W_up = jnp.concatenate([dec0_Ws[:C2], dec0_Wn[:C2]], axis=1)
    gAB = _gather_rows(_proj(h, W_up), pmap1g, E1p)
    y = _lin(skip1, dec0_Wn[C2:], gAB[:, C1:])
    y = jnp.pad(y, ((0, 0), (0, _rup(C1, 128) - C1)))
    h = _fin(skip1, dec0_Ws[C2:], gAB[:, :C1], _gather4(y, adj1f, E1p),
             dec0_b, dec0_g, dec0_be)

    C0 = dec1_Ws.shape[1]
    W_up0 = jnp.concatenate([dec1_Ws[:C1], dec1_Wn[:C1]], axis=1)
    gAB0 = _gather_rows(_proj(h, W_up0), pmap0g, E0p)
    y0 = _lin(skip0, dec1_Wn[C1:], gAB0[:, C0:])
    h = _fin(skip0, dec1_Ws[C1:], gAB0[:, :C0], _gather4(y0, adj0f, E0p),
             dec1_b, dec1_g, dec1_be)

    out = _head(h, head_W1, head_b1, head_W2, head_b2)
    return out[:E0, 0]


# speedup vs baseline: 1.2496x; 1.0081x over previous
import functools
import jax
import jax.numpy as jnp
from jax import lax
from jax.experimental import pallas as pl
from jax.experimental.pallas import tpu as pltpu
from jax.experimental.pallas import tpu_sc as plsc

RB = 512
SC_CH = 128
SC_NW = 32


def _ln_gelu(h, g, be):
    mu = jnp.mean(h, axis=-1, keepdims=True)
    var = jnp.mean((h - mu) ** 2, axis=-1, keepdims=True)
    return jax.nn.gelu((h - mu) / jnp.sqrt(var + 1e-5) * g + be)


def _row_spec(C):
    return pl.BlockSpec((RB, C), lambda i: (i, 0))


def _full_spec(shape):
    return pl.BlockSpec(shape, lambda i: (0,) * len(shape))


def _stem_body(x_ref, W1_ref, b1_ref, g1_ref, be1_ref, W2_ref, b2_ref, g2_ref,
               be2_ref, o_ref):
    h = jnp.dot(x_ref[...], W1_ref[...], preferred_element_type=jnp.float32)
    h = _ln_gelu(h + b1_ref[...], g1_ref[...], be1_ref[...])
    h = jnp.dot(h, W2_ref[...], preferred_element_type=jnp.float32)
    o_ref[...] = _ln_gelu(h + b2_ref[...], g2_ref[...], be2_ref[...])


def _stem(x, W1, b1, g1, be1, W2, b2, g2, be2):
    N = x.shape[0]
    Co = W2.shape[1]
    args = [x, W1, b1.reshape(1, -1), g1.reshape(1, -1), be1.reshape(1, -1),
            W2, b2.reshape(1, -1), g2.reshape(1, -1), be2.reshape(1, -1)]
    return pl.pallas_call(
        _stem_body,
        grid=(N // RB,),
        in_specs=[_row_spec(x.shape[1])] + [_full_spec(a.shape) for a in args[1:]],
        out_specs=_row_spec(Co),
        out_shape=jax.ShapeDtypeStruct((N, Co), jnp.float32),
    )(*args)


def _cb_body(x_ref, n4_ref, Ws_ref, Wn_ref, b_ref, g_ref, be_ref, o_ref):
    nbrm = 0.25 * _sum4(n4_ref[...], x_ref.shape[1])
    h = jnp.dot(x_ref[...], Ws_ref[...], preferred_element_type=jnp.float32)
    h = h + jnp.dot(nbrm, Wn_ref[...], preferred_element_type=jnp.float32)
    o_ref[...] = _ln_gelu(h + b_ref[...], g_ref[...], be_ref[...])


def _cb(x, nbr4, Ws, Wn, b, g, be):
    N = x.shape[0]
    Co = Ws.shape[1]
    args = [x, nbr4, Ws, Wn, b.reshape(1, -1), g.reshape(1, -1), be.reshape(1, -1)]
    return pl.pallas_call(
        _cb_body,
        grid=(N // RB,),
        in_specs=[_row_spec(x.shape[1]), _row_spec(nbr4.shape[1])]
        + [_full_spec(a.shape) for a in args[2:]],
        out_specs=_row_spec(Co),
        out_shape=jax.ShapeDtypeStruct((N, Co), jnp.float32),
    )(*args)


def _proj_body(x_ref, W_ref, o_ref):
    o_ref[...] = jnp.dot(x_ref[...], W_ref[...], preferred_element_type=jnp.float32)


def _proj(x, W):
    N = x.shape[0]
    Co = W.shape[1]
    return pl.pallas_call(
        _proj_body,
        grid=(N // RB,),
        in_specs=[_row_spec(x.shape[1]), _full_spec(W.shape)],
        out_specs=_row_spec(Co),
        out_shape=jax.ShapeDtypeStruct((N, Co), jnp.float32),
    )(x, W)


def _lin_body(x_ref, W_ref, a_ref, o_ref):
    o_ref[...] = a_ref[...] + jnp.dot(x_ref[...], W_ref[...],
                                      preferred_element_type=jnp.float32)


def _lin(x, W, a):
    N = x.shape[0]
    Co = W.shape[1]
    return pl.pallas_call(
        _lin_body,
        grid=(N // RB,),
        in_specs=[_row_spec(x.shape[1]), _full_spec(W.shape), _row_spec(Co)],
        out_specs=_row_spec(Co),
        out_shape=jax.ShapeDtypeStruct((N, Co), jnp.float32),
    )(x, W, a)


def _sum4(n4, Co):
    Cp = n4.shape[1] // 4
    return (n4[:, :Co] + n4[:, Cp:Cp + Co] + n4[:, 2 * Cp:2 * Cp + Co]
            + n4[:, 3 * Cp:3 * Cp + Co])


def _cbp_body(x_ref, n4_ref, Ws_ref, b_ref, g_ref, be_ref, o_ref):
    h = jnp.dot(x_ref[...], Ws_ref[...], preferred_element_type=jnp.float32)
    h = h + 0.25 * _sum4(n4_ref[...], Ws_ref.shape[1]) + b_ref[...]
    o_ref[...] = _ln_gelu(h, g_ref[...], be_ref[...])


def _cbp(x, nbr4, Ws, b, g, be):
    N = x.shape[0]
    Co = Ws.shape[1]
    args = [x, nbr4, Ws, b.reshape(1, -1), g.reshape(1, -1), be.reshape(1, -1)]
    return pl.pallas_call(
        _cbp_body,
        grid=(N // RB,),
        in_specs=[_row_spec(x.shape[1]), _row_spec(nbr4.shape[1])]
        + [_full_spec(a.shape) for a in args[2:]],
        out_specs=_row_spec(Co),
        out_shape=jax.ShapeDtypeStruct((N, Co), jnp.float32),
    )(*args)


def _fin_body(x_ref, W_ref, a1_ref, n4_ref, b_ref, g_ref, be_ref, o_ref):
    Co = W_ref.shape[1]
    nbrm = 0.25 * _sum4(n4_ref[...], Co)
    h = jnp.dot(x_ref[...], W_ref[...], preferred_element_type=jnp.float32)
    h = h + a1_ref[...] + nbrm + b_ref[...]
    o_ref[...] = _ln_gelu(h, g_ref[...], be_ref[...])


def _fin(x, W, a1, nbr4, b, g, be):
    N = x.shape[0]
    Co = W.shape[1]
    args = [x, W, a1, nbr4, b.reshape(1, -1), g.reshape(1, -1), be.reshape(1, -1)]
    return pl.pallas_call(
        _fin_body,
        grid=(N // RB,),
        in_specs=[_row_spec(x.shape[1]), _full_spec(W.shape), _row_spec(Co),
                  _row_spec(nbr4.shape[1])] + [_full_spec(a.shape) for a in args[4:]],
        out_specs=_row_spec(Co),
        out_shape=jax.ShapeDtypeStruct((N, Co), jnp.float32),
    )(*args)


def _head_body(x_ref, W1_ref, b1_ref, W2_ref, b2_ref, o_ref):
    h = jnp.dot(x_ref[...], W1_ref[...], preferred_element_type=jnp.float32)
    h = jax.nn.gelu(h + b1_ref[...])
    o_ref[...] = jnp.dot(h, W2_ref[...],
                         preferred_element_type=jnp.float32) + b2_ref[...]


def _head(x, W1, b1, W2, b2):
    N = x.shape[0]
    args = [x, W1, b1.reshape(1, -1), W2, b2.reshape(1, -1)]
    return pl.pallas_call(
        _head_body,
        grid=(N // RB,),
        in_specs=[_row_spec(x.shape[1])] + [_full_spec(a.shape) for a in args[1:]],
        out_specs=_row_spec(1),
        out_shape=jax.ShapeDtypeStruct((N, 1), jnp.float32),
    )(*args)


def _sc_gather(table, idx):
    C = table.shape[1]
    gran = 2 * SC_NW * SC_CH
    Mp = _rup(idx.shape[0], gran)
    idx = jnp.pad(idx, (0, Mp + SC_NW * SC_CH - idx.shape[0]))
    per_w = Mp // SC_NW
    pairs = per_w // (2 * SC_CH)

    @functools.partial(
        pl.kernel,
        mesh=plsc.VectorSubcoreMesh(core_axis_name="c", subcore_axis_name="s"),
        out_type=jax.ShapeDtypeStruct((Mp, C), jnp.float32),
        scratch_types=[
            pltpu.VMEM((SC_CH,), jnp.int32),
            pltpu.VMEM((SC_CH,), jnp.int32),
            pltpu.VMEM((SC_CH, C), jnp.float32),
            pltpu.VMEM((SC_CH, C), jnp.float32),
            pltpu.SemaphoreType.DMA,
            pltpu.SemaphoreType.DMA,
        ],
    )
    def k(table_hbm, idx_hbm, out_hbm, ia, ib, ra, rb, sema, semb):
        wid = lax.axis_index("s") * 2 + lax.axis_index("c")
        base0 = wid * per_w

        pltpu.sync_copy(idx_hbm.at[pl.ds(base0, SC_CH)], ia)
        pltpu.async_copy(table_hbm.at[ia], ra, sema)

        def body(jj, carry):
            b0 = base0 + 2 * jj * SC_CH
            pltpu.sync_copy(idx_hbm.at[pl.ds(b0 + SC_CH, SC_CH)], ib)
            pltpu.async_copy(table_hbm.at[ib], rb, semb)
            pltpu.make_async_copy(table_hbm.at[ia], ra, sema).wait()
            pltpu.sync_copy(ra, out_hbm.at[pl.ds(b0, SC_CH)])
            pltpu.sync_copy(idx_hbm.at[pl.ds(b0 + 2 * SC_CH, SC_CH)], ia)
            pltpu.async_copy(table_hbm.at[ia], ra, sema)
            pltpu.make_async_copy(table_hbm.at[ib], rb, semb).wait()
            pltpu.sync_copy(rb, out_hbm.at[pl.ds(b0 + SC_CH, SC_CH)])
            return carry

        lax.fori_loop(0, pairs, body, 0)
        pltpu.make_async_copy(table_hbm.at[ia], ra, sema).wait()

    return k(table, idx)


def _gather4(x, adj_flat, n_real):
    g = _sc_gather(x, adj_flat)
    C = x.shape[1]
    return g[:4 * n_real].reshape(n_real, 4 * C)


def _gather_rows(x, idx, n_real):
    return _sc_gather(x, idx)[:n_real]


def _seg_mean(h, pmap, n_out):
    sums = jax.ops.segment_sum(h, pmap, num_segments=n_out)
    cnt = jax.ops.segment_sum(jnp.ones((h.shape[0],), h.dtype), pmap,
                              num_segments=n_out)
    return sums / jnp.maximum(cnt, 1.0)[:, None]


def _pad_rows(a, n, val=0):
    return jnp.pad(a, ((0, n - a.shape[0]),) + ((0, 0),) * (a.ndim - 1),
                   constant_values=val)


def _rup(n, m=RB):
    return ((n + m - 1) // m) * m


_SC_M = SC_NW * SC_CH


def _pad_idx(a, val=0):
    return _pad_rows(a.reshape(-1, 1), _rup(a.shape[0], _SC_M), val).reshape(-1)


def kernel(edge_features, slot_adj_l0, slot_adj_l1, slot_adj_l2, pool_map_l0,
           pool_map_l1, stem_W1, stem_b1, stem_g1, stem_be1, stem_W2, stem_b2,
           stem_g2, stem_be2, enc0_Ws, enc0_Wn, enc0_b, enc0_g, enc0_be,
           enc1_Ws, enc1_Wn, enc1_b, enc1_g, enc1_be, bot_Ws, bot_Wn, bot_b,
           bot_g, bot_be, dec0_Ws, dec0_Wn, dec0_b, dec0_g, dec0_be, dec1_Ws,
           dec1_Wn, dec1_b, dec1_g, dec1_be, head_W1, head_b1, head_W2,
           head_b2):
    E0, E1, E2 = edge_features.shape[0], slot_adj_l1.shape[0], slot_adj_l2.shape[0]
    C1, C2 = enc1_Ws.shape[1], bot_Ws.shape[1]
    E0p, E1p, E2p = _rup(E0), _rup(E1), _rup(E2)

    x = _pad_rows(edge_features, E0p)
    adj0f = _pad_idx(_pad_rows(slot_adj_l0, E0p).reshape(-1))
    adj1f = _pad_idx(_pad_rows(slot_adj_l1, E1p).reshape(-1))
    adj2f = _pad_idx(_pad_rows(slot_adj_l2, E2p).reshape(-1))
    pmap0 = _pad_rows(pool_map_l0.reshape(-1, 1), E0p, E1).reshape(-1)
    pmap1 = _pad_rows(pool_map_l1.reshape(-1, 1), E1p, E2).reshape(-1)
    pmap0g = _pad_idx(pmap0)
    pmap1g = _pad_idx(pmap1)

    h = _stem(x, stem_W1, stem_b1, stem_g1, stem_be1, stem_W2, stem_b2,
              stem_g2, stem_be2)
    h = _cb(h, _gather4(h, adj0f, E0p), enc0_Ws, enc0_Wn, enc0_b, enc0_g, enc0_be)
    skip0 = h
    h = _seg_mean(h, pmap0, E1p)
    h = _cb(h, _gather4(h, adj1f, E1p), enc1_Ws, enc1_Wn, enc1_b, enc1_g, enc1_be)
    skip1 = h
    h = _seg_mean(h, pmap1, E2p)
    hp = jnp.pad(h, ((0, 0), (0, _rup(h.shape[1], 128) - h.shape[1])))
    h = _cb(h, _gather4(hp, adj2f, E2p), bot_Ws, bot_Wn, bot_b, bot_g, bot_be)

    W_up = jnp.concatenate([dec0_Ws[:C2], dec0_Wn[:C2]], axis=1)
    gAB = _gather_rows(_proj(h, W_up), pmap1g, E1p)
    y = _lin(skip1, dec0_Wn[C2:], gAB[:, C1:])
    y = jnp.pad(y, ((0, 0), (0, _rup(C1, 128) - C1)))
    h = _fin(skip1, dec0_Ws[C2:], gAB[:, :C1], _gather4(y, adj1f, E1p),
             dec0_b, dec0_g, dec0_be)

    C0 = dec1_Ws.shape[1]
    W_up0 = jnp.concatenate([dec1_Ws[:C1], dec1_Wn[:C1]], axis=1)
    gAB0 = _gather_rows(_proj(h, W_up0), pmap0g, E0p)
    y0 = _lin(skip0, dec1_Wn[C1:], gAB0[:, C0:])
    h = _fin(skip0, dec1_Ws[C1:], gAB0[:, :C0], _gather4(y0, adj0f, E0p),
             dec1_b, dec1_g, dec1_be)

    out = _head(h, head_W1, head_b1, head_W2, head_b2)
    return out[:E0, 0]
